# Initial kernel scaffold; baseline (speedup 1.0000x reference)
import functools
import jax
import jax.numpy as jnp
from jax import lax
from jax.experimental import pallas as pl
from jax.experimental.pallas import tpu as pltpu
from jax.experimental.pallas import tpu_sc as plsc

NC, NS, LANES = 2, 16, 16
NW = NC * NS
CHK = 128

_MESH = dict(core_axis_name="c", subcore_axis_name="s", num_cores=NC,
             num_subcores=NS)


def _sc_deg(dst, ew, zero16, n_nodes):
    n_edges = dst.shape[0]
    n_chunks = n_edges // CHK

    @functools.partial(
        pl.kernel,
        out_type=jax.ShapeDtypeStruct((NC, n_nodes, LANES), jnp.float32),
        mesh=plsc.VectorSubcoreMesh(**_MESH),
        scratch_types=[
            pltpu.VMEM_SHARED((n_nodes, LANES), jnp.float32),
            pltpu.VMEM((CHK,), jnp.int32),
            pltpu.VMEM((CHK,), jnp.float32),
            pltpu.VMEM((CHK, LANES), jnp.float32),
            pltpu.SemaphoreType.DMA,
        ],
    )
    def deg_kernel(dst_hbm, ew_hbm, zero_hbm, degp_hbm, shared, idx_v, ew_v,
                   rows_v, sem):
        cid = lax.axis_index("c")
        sid = lax.axis_index("s")
        wid = sid * NC + cid
        rpt = n_nodes // NS
        row0 = sid * rpt
        pltpu.async_copy(zero_hbm.at[pl.ds(row0, rpt)],
                         shared.at[pl.ds(row0, rpt)], sem).wait()
        plsc.subcore_barrier()

        @pl.loop(wid, n_chunks, step=NW)
        def _(chk):
            base = chk * CHK
            pltpu.async_copy(dst_hbm.at[pl.ds(base, CHK)], idx_v, sem).wait()
            pltpu.async_copy(ew_hbm.at[pl.ds(base, CHK)], ew_v, sem).wait()

            @pl.loop(0, CHK)
            def _(e):
                rows_v[e] = plsc.load_gather(ew_v, [lax.broadcast(e, (LANES,))])

            pltpu.sync_copy(rows_v, shared.at[idx_v], add=True)

        plsc.subcore_barrier()
        pltpu.async_copy(shared.at[pl.ds(row0, rpt)],
                         degp_hbm.at[cid, pl.ds(row0, rpt)], sem).wait()

    return deg_kernel(dst, ew, zero16)


def _sc_msgs(src, dst, ew, g, zero_d, n_nodes, d_out):
    n_edges = src.shape[0]
    n_chunks = n_edges // CHK
    n_vec = d_out // LANES

    @functools.partial(
        pl.kernel,
        out_type=jax.ShapeDtypeStruct((NC, n_nodes, d_out), jnp.float32),
        mesh=plsc.VectorSubcoreMesh(**_MESH),
        scratch_types=[
            pltpu.VMEM_SHARED((n_nodes, d_out), jnp.float32),
            pltpu.VMEM((CHK,), jnp.int32),
            pltpu.VMEM((CHK,), jnp.int32),
            pltpu.VMEM((CHK,), jnp.float32),
            pltpu.VMEM((CHK, d_out), jnp.float32),
            pltpu.SemaphoreType.DMA,
            pltpu.SemaphoreType.DMA,
        ],
    )
    def msg_kernel(src_hbm, dst_hbm, ew_hbm, g_hbm, zero_hbm, accp_hbm,
                   acc_sh, sidx, didx, ew_v, rows_v, sem, gsem):
        cid = lax.axis_index("c")
        sid = lax.axis_index("s")
        wid = sid * NC + cid
        rpt = n_nodes // NS
        row0 = sid * rpt
        pltpu.async_copy(zero_hbm.at[pl.ds(row0, rpt)],
                         acc_sh.at[pl.ds(row0, rpt)], sem).wait()
        plsc.subcore_barrier()

        @pl.loop(wid, n_chunks, step=NW)
        def _(chk):
            base = chk * CHK
            pltpu.async_copy(src_hbm.at[pl.ds(base, CHK)], sidx, sem).wait()
            pltpu.async_copy(ew_hbm.at[pl.ds(base, CHK)], ew_v, sem).wait()
            pltpu.async_copy(dst_hbm.at[pl.ds(base, CHK)], didx, sem).wait()
            pltpu.async_copy(g_hbm.at[sidx], rows_v, gsem).wait()

            @pl.loop(0, CHK)
            def _(e):
                sp = plsc.load_gather(ew_v, [lax.broadcast(e, (LANES,))])
                for j in range(n_vec):
                    sl = pl.ds(j * LANES, LANES)
                    rows_v[e, sl] = rows_v[e, sl] * sp

            pltpu.sync_copy(rows_v, acc_sh.at[didx], add=True)

        plsc.subcore_barrier()
        pltpu.async_copy(acc_sh.at[pl.ds(row0, rpt)],
                         accp_hbm.at[cid, pl.ds(row0, rpt)], sem).wait()

    return msg_kernel(src, dst, ew, g, zero_d)


def _tc_g(x, W, degp, block_n):
    n, d_in = x.shape
    d_out = W.shape[1]

    def body(x_ref, w_ref, degp_ref, g_ref):
        h = jnp.dot(x_ref[...], w_ref[...], preferred_element_type=jnp.float32)
        deg = degp_ref[0] + degp_ref[1] + 1.0
        dis = jnp.where(deg > 0, lax.rsqrt(jnp.maximum(deg, 1e-38)), 0.0)
        g_ref[...] = h * dis[:, 0:1]

    return pl.pallas_call(
        body,
        grid=(n // block_n,),
        in_specs=[
            pl.BlockSpec((block_n, d_in), lambda i: (i, 0)),
            pl.BlockSpec((d_in, d_out), lambda i: (0, 0)),
            pl.BlockSpec((NC, block_n, LANES), lambda i: (0, i, 0)),
        ],
        out_specs=pl.BlockSpec((block_n, d_out), lambda i: (i, 0)),
        out_shape=jax.ShapeDtypeStruct((n, d_out), jnp.float32),
    )(x, W, degp)


def _tc_out(accp, g, degp, b2d, block_n):
    n, d_out = g.shape

    def body(accp_ref, g_ref, degp_ref, b_ref, o_ref):
        s = accp_ref[0] + accp_ref[1] + g_ref[...]
        deg = degp_ref[0] + degp_ref[1] + 1.0
        dis = jnp.where(deg > 0, lax.rsqrt(jnp.maximum(deg, 1e-38)), 0.0)
        z = s * dis[:, 0:1] + b_ref[...]
        m = jnp.max(z, axis=-1, keepdims=True)
        lse = m + jnp.log(jnp.sum(jnp.exp(z - m), axis=-1, keepdims=True))
        o_ref[...] = z - lse

    return pl.pallas_call(
        body,
        grid=(n // block_n,),
        in_specs=[
            pl.BlockSpec((NC, block_n, d_out), lambda i: (0, i, 0)),
            pl.BlockSpec((block_n, d_out), lambda i: (i, 0)),
            pl.BlockSpec((NC, block_n, LANES), lambda i: (0, i, 0)),
            pl.BlockSpec((1, d_out), lambda i: (0, 0)),
        ],
        out_specs=pl.BlockSpec((block_n, d_out), lambda i: (i, 0)),
        out_shape=jax.ShapeDtypeStruct((n, d_out), jnp.float32),
    )(accp, g, degp, b2d)


@jax.jit
def kernel(x, edge_index, edge_weight, W, b):
    n_nodes, _ = x.shape
    d_out = W.shape[1]
    src = edge_index[0].astype(jnp.int32)
    dst = edge_index[1].astype(jnp.int32)
    ew = edge_weight.astype(jnp.float32)

    zero16 = jnp.zeros((n_nodes, LANES), jnp.float32)
    zero_d = jnp.zeros((n_nodes, d_out), jnp.float32)

    degp = _sc_deg(dst, ew, zero16, n_nodes)
    g = _tc_g(x, W, degp, block_n=2000)
    accp = _sc_msgs(src, dst, ew, g, zero_d, n_nodes, d_out)
    return _tc_out(accp, g, degp, jnp.reshape(b, (1, d_out)), block_n=2000)


# speedup vs baseline: 13.4952x; 13.4952x over previous
import dataclasses
import functools
import jax
import jax.numpy as jnp
from jax import lax
from jax.experimental import pallas as pl
from jax.experimental.pallas import tpu as pltpu
from jax.experimental.pallas import tpu_sc as plsc

NC, NS, LANES = 2, 16, 16
NW = NC * NS
CHK = 128
RC = 400

_MESH = dict(core_axis_name="c", subcore_axis_name="s", num_cores=NC,
             num_subcores=NS)

_SC_PARAMS = dataclasses.replace(pltpu.CompilerParams(),
                                 needs_layout_passes=False)
_SC_PARAMS_LINEAR = dataclasses.replace(_SC_PARAMS, use_tc_tiling_on_sc=False)


def _sc_deg(dst, ew, zero16, n_nodes):
    n_edges = dst.shape[0]
    n_chunks = n_edges // CHK

    @functools.partial(
        pl.kernel,
        out_type=jax.ShapeDtypeStruct((NC, n_nodes, LANES), jnp.float32),
        mesh=plsc.VectorSubcoreMesh(**_MESH),
        compiler_params=_SC_PARAMS_LINEAR,
        scratch_types=[
            pltpu.VMEM_SHARED((n_nodes, LANES), jnp.float32),
            pltpu.VMEM((CHK,), jnp.int32),
            pltpu.VMEM((CHK,), jnp.float32),
            pltpu.VMEM((CHK, LANES), jnp.float32),
            pltpu.SemaphoreType.DMA,
        ],
    )
    def deg_kernel(dst_hbm, ew_hbm, zero_hbm, degp_hbm, shared, idx_v, ew_v,
                   rows_v, sem):
        cid = lax.axis_index("c")
        sid = lax.axis_index("s")
        wid = sid * NC + cid
        n_rchunks = n_nodes // RC

        @pl.loop(sid, n_rchunks, step=NS)
        def _(rc):
            r0 = pl.multiple_of(rc * RC, RC)
            pltpu.async_copy(zero_hbm.at[pl.ds(r0, RC)],
                             shared.at[pl.ds(r0, RC)], sem).wait()

        plsc.subcore_barrier()

        @pl.loop(wid, n_chunks, step=NW)
        def _(chk):
            base = pl.multiple_of(chk * CHK, CHK)
            pltpu.async_copy(dst_hbm.at[pl.ds(base, CHK)], idx_v, sem).wait()
            pltpu.async_copy(ew_hbm.at[pl.ds(base, CHK)], ew_v, sem).wait()

            @pl.loop(0, CHK)
            def _(e):
                rows_v[e] = plsc.load_gather(ew_v, [lax.broadcast(e, (LANES,))])

            pltpu.sync_copy(rows_v, shared.at[idx_v], add=True)

        plsc.subcore_barrier()

        @pl.loop(sid, n_rchunks, step=NS)
        def _(rc):
            r0 = pl.multiple_of(rc * RC, RC)
            pltpu.async_copy(shared.at[pl.ds(r0, RC)],
                             degp_hbm.at[cid, pl.ds(r0, RC)], sem).wait()

    return deg_kernel(dst, ew, zero16)


def _sc_msgs(src, dst, ew, g, zero_d, n_nodes, d_out):
    n_edges = src.shape[0]
    n_chunks = n_edges // CHK
    n_vec = d_out // LANES

    @functools.partial(
        pl.kernel,
        out_type=jax.ShapeDtypeStruct((NC, n_nodes, d_out), jnp.float32),
        mesh=plsc.VectorSubcoreMesh(**_MESH),
        compiler_params=_SC_PARAMS,
        scratch_types=[
            pltpu.VMEM_SHARED((n_nodes, d_out), jnp.float32),
            pltpu.VMEM((CHK,), jnp.int32),
            pltpu.VMEM((CHK,), jnp.int32),
            pltpu.VMEM((CHK,), jnp.float32),
            pltpu.VMEM((CHK, d_out), jnp.float32),
            pltpu.SemaphoreType.DMA,
            pltpu.SemaphoreType.DMA,
        ],
    )
    def msg_kernel(src_hbm, dst_hbm, ew_hbm, g_hbm, zero_hbm, accp_hbm,
                   acc_sh, sidx, didx, ew_v, rows_v, sem, gsem):
        cid = lax.axis_index("c")
        sid = lax.axis_index("s")
        wid = sid * NC + cid
        n_rchunks = n_nodes // RC

        @pl.loop(sid, n_rchunks, step=NS)
        def _(rc):
            r0 = pl.multiple_of(rc * RC, RC)
            pltpu.async_copy(zero_hbm.at[pl.ds(r0, RC)],
                             acc_sh.at[pl.ds(r0, RC)], sem).wait()

        plsc.subcore_barrier()

        @pl.loop(wid, n_chunks, step=NW)
        def _(chk):
            base = pl.multiple_of(chk * CHK, CHK)
            pltpu.async_copy(src_hbm.at[pl.ds(base, CHK)], sidx, sem).wait()
            pltpu.async_copy(ew_hbm.at[pl.ds(base, CHK)], ew_v, sem).wait()
            pltpu.async_copy(dst_hbm.at[pl.ds(base, CHK)], didx, sem).wait()
            pltpu.async_copy(g_hbm.at[sidx], rows_v, gsem).wait()

            @pl.loop(0, CHK)
            def _(e):
                sp = plsc.load_gather(ew_v, [lax.broadcast(e, (LANES,))])
                for j in range(n_vec):
                    sl = pl.ds(j * LANES, LANES)
                    rows_v[e, sl] = rows_v[e, sl] * sp

            pltpu.sync_copy(rows_v, acc_sh.at[didx], add=True)

        plsc.subcore_barrier()

        @pl.loop(sid, n_rchunks, step=NS)
        def _(rc):
            r0 = pl.multiple_of(rc * RC, RC)
            pltpu.async_copy(acc_sh.at[pl.ds(r0, RC)],
                             accp_hbm.at[cid, pl.ds(r0, RC)], sem).wait()

    return msg_kernel(src, dst, ew, g, zero_d)


def _tc_g(x, W, degp, block_n):
    n, d_in = x.shape
    d_out = W.shape[1]

    def body(x_ref, w_ref, degp_ref, g_ref):
        h = jnp.dot(x_ref[...], w_ref[...], preferred_element_type=jnp.float32)
        deg = degp_ref[0] + degp_ref[1] + 1.0
        dis = jnp.where(deg > 0, lax.rsqrt(jnp.maximum(deg, 1e-38)), 0.0)
        g_ref[...] = h * dis[:, 0:1]

    return pl.pallas_call(
        body,
        grid=(n // block_n,),
        in_specs=[
            pl.BlockSpec((block_n, d_in), lambda i: (i, 0)),
            pl.BlockSpec((d_in, d_out), lambda i: (0, 0)),
            pl.BlockSpec((NC, block_n, LANES), lambda i: (0, i, 0)),
        ],
        out_specs=pl.BlockSpec((block_n, d_out), lambda i: (i, 0)),
        out_shape=jax.ShapeDtypeStruct((n, d_out), jnp.float32),
    )(x, W, degp)


def _tc_out(accp, g, degp, b2d, block_n):
    n, d_out = g.shape

    def body(accp_ref, g_ref, degp_ref, b_ref, o_ref):
        s = accp_ref[0] + accp_ref[1] + g_ref[...]
        deg = degp_ref[0] + degp_ref[1] + 1.0
        dis = jnp.where(deg > 0, lax.rsqrt(jnp.maximum(deg, 1e-38)), 0.0)
        z = s * dis[:, 0:1] + b_ref[...]
        m = jnp.max(z, axis=-1, keepdims=True)
        lse = m + jnp.log(jnp.sum(jnp.exp(z - m), axis=-1, keepdims=True))
        o_ref[...] = z - lse

    return pl.pallas_call(
        body,
        grid=(n // block_n,),
        in_specs=[
            pl.BlockSpec((NC, block_n, d_out), lambda i: (0, i, 0)),
            pl.BlockSpec((block_n, d_out), lambda i: (i, 0)),
            pl.BlockSpec((NC, block_n, LANES), lambda i: (0, i, 0)),
            pl.BlockSpec((1, d_out), lambda i: (0, 0)),
        ],
        out_specs=pl.BlockSpec((block_n, d_out), lambda i: (i, 0)),
        out_shape=jax.ShapeDtypeStruct((n, d_out), jnp.float32),
    )(accp, g, degp, b2d)


@jax.jit
def kernel(x, edge_index, edge_weight, W, b):
    n_nodes, _ = x.shape
    d_out = W.shape[1]
    src = edge_index[0].astype(jnp.int32)
    dst = edge_index[1].astype(jnp.int32)
    ew = edge_weight.astype(jnp.float32)

    zero16 = jnp.zeros((n_nodes, LANES), jnp.float32)
    zero_d = jnp.zeros((n_nodes, d_out), jnp.float32)

    degp = _sc_deg(dst, ew, zero16, n_nodes)
    g = _tc_g(x, W, degp, block_n=2000)
    accp = _sc_msgs(src, dst, ew, g, zero_d, n_nodes, d_out)
    return _tc_out(accp, g, degp, jnp.reshape(b, (1, d_out)), block_n=2000)
